# baseline (speedup 1.0000x reference)
import functools

import jax
import jax.numpy as jnp
from jax import lax
from jax.experimental import pallas as pl
from jax.experimental.pallas import tpu as pltpu
from jax.experimental.pallas import tpu_sc as plsc

VOCAB = 100000
EMBED_DIM = 128
BATCH = 4096
SEQ_LEN = 50

_info = plsc.get_sparse_core_info()
_NC, _NS = _info.num_cores, _info.num_subcores
_NW = _NC * _NS
_TOTAL = BATCH * SEQ_LEN
_PER_W = _TOTAL // _NW
_SUB = 64
_SPS = 5
_SLOT = _SUB * _SPS
_N_SLOTS = _PER_W // _SLOT
_N_SUB = _PER_W // _SUB
_NBUF = 2
_N_GROUPS = _N_SLOTS // _NBUF

_mesh = plsc.VectorSubcoreMesh(core_axis_name="c", subcore_axis_name="s")


@functools.partial(
    pl.kernel,
    out_type=jax.ShapeDtypeStruct((_TOTAL, EMBED_DIM), jnp.float32),
    mesh=_mesh,
    scratch_types=[
        pltpu.VMEM((_N_SUB, _SUB), jnp.int32),
        [pltpu.VMEM((_SLOT, EMBED_DIM), jnp.float32) for _ in range(_NBUF)],
        [pltpu.SemaphoreType.DMA for _ in range(_NBUF)],
        [pltpu.SemaphoreType.DMA for _ in range(_NBUF)],
    ],
)
def _embed_gather(idx_hbm, table_hbm, out_hbm, idx_v, bufs, gsems, osems):
    wid = lax.axis_index("s") * _NC + lax.axis_index("c")
    base = wid * _PER_W
    pltpu.sync_copy(idx_hbm.at[wid], idx_v)

    def start_gather(t, b):
        for k in range(_SPS):
            pltpu.async_copy(
                table_hbm.at[idx_v.at[t * _SPS + k]],
                bufs[b].at[pl.ds(k * _SUB, _SUB)],
                gsems[b],
            )

    def wait_gather(b):
        pltpu.make_async_copy(
            table_hbm.at[pl.ds(0, _SLOT)], bufs[b], gsems[b]
        ).wait()

    def out_slice(t):
        return out_hbm.at[pl.ds(base + t * _SLOT, _SLOT)]

    def start_out(t, b):
        pltpu.async_copy(bufs[b], out_slice(t), osems[b])

    def wait_out(t, b):
        pltpu.make_async_copy(bufs[b], out_slice(t), osems[b]).wait()

    for b in range(_NBUF):
        start_gather(b, b)

    @pl.loop(0, _N_GROUPS - 1)
    def _grp(g):
        t0 = g * _NBUF
        for b in range(_NBUF):
            wait_gather(b)
            start_out(t0 + b, b)
        for b in range(_NBUF):
            wait_out(t0 + b, b)
            start_gather(t0 + b + _NBUF, b)

    t0 = (_N_GROUPS - 1) * _NBUF
    for b in range(_NBUF):
        wait_gather(b)
        start_out(t0 + b, b)
    for b in range(_NBUF):
        wait_out(t0 + b, b)


def kernel(numericalized_tokens, embedding_table):
    idx = numericalized_tokens.astype(jnp.int32).T.reshape(_NW, _N_SUB, _SUB)
    out = _embed_gather(idx, embedding_table)
    return out.reshape(SEQ_LEN, BATCH, EMBED_DIM).transpose(1, 0, 2)


# speedup vs baseline: 1.0594x; 1.0594x over previous
import functools

import jax
import jax.numpy as jnp
from jax import lax
from jax.experimental import pallas as pl
from jax.experimental.pallas import tpu as pltpu
from jax.experimental.pallas import tpu_sc as plsc

VOCAB = 100000
EMBED_DIM = 128
BATCH = 4096
SEQ_LEN = 50

_info = plsc.get_sparse_core_info()
_NC, _NS = _info.num_cores, _info.num_subcores
_NW = _NC * _NS
_TOTAL = BATCH * SEQ_LEN
_CHUNK = 64
_PER_W = _TOTAL // _NW
_N_CHUNKS = _PER_W // _CHUNK

_NBUF = 10
_N_GROUPS = _N_CHUNKS // _NBUF

_mesh = plsc.VectorSubcoreMesh(core_axis_name="c", subcore_axis_name="s")


@functools.partial(
    pl.kernel,
    out_type=jax.ShapeDtypeStruct((_TOTAL, EMBED_DIM), jnp.float32),
    mesh=_mesh,
    scratch_types=[
        pltpu.VMEM((_N_CHUNKS, _CHUNK), jnp.int32),
        [pltpu.VMEM((_CHUNK, EMBED_DIM), jnp.float32) for _ in range(_NBUF)],
        [pltpu.SemaphoreType.DMA for _ in range(_NBUF)],
        [pltpu.SemaphoreType.DMA for _ in range(_NBUF)],
    ],
)
def _embed_gather(idx_hbm, table_hbm, out_hbm, idx_v, bufs, gsems, osems):
    wid = lax.axis_index("s") * _NC + lax.axis_index("c")
    base = wid * _PER_W
    pltpu.sync_copy(idx_hbm.at[wid], idx_v)

    def start_gather(j, b):
        pltpu.async_copy(table_hbm.at[idx_v.at[j]], bufs[b], gsems[b])

    def wait_gather(j, b):
        pltpu.make_async_copy(table_hbm.at[idx_v.at[j]], bufs[b], gsems[b]).wait()

    def out_slice(j):
        return out_hbm.at[pl.ds(base + j * _CHUNK, _CHUNK)]

    def start_out(j, b):
        pltpu.async_copy(bufs[b], out_slice(j), osems[b])

    def wait_out(j, b):
        pltpu.make_async_copy(bufs[b], out_slice(j), osems[b]).wait()

    for b in range(_NBUF):
        start_gather(b, b)

    @pl.loop(0, _N_GROUPS - 1)
    def _grp(g):
        j0 = g * _NBUF
        for b in range(_NBUF):
            wait_gather(j0 + b, b)
            start_out(j0 + b, b)
        for b in range(_NBUF):
            wait_out(j0 + b, b)
            start_gather(j0 + b + _NBUF, b)

    j0 = (_N_GROUPS - 1) * _NBUF
    for b in range(_NBUF):
        wait_gather(j0 + b, b)
        start_out(j0 + b, b)
    for b in range(_NBUF):
        wait_out(j0 + b, b)


def kernel(numericalized_tokens, embedding_table):
    idx = numericalized_tokens.astype(jnp.int32).T.reshape(_NW, _N_CHUNKS, _CHUNK)
    out = _embed_gather(idx, embedding_table)
    return out.reshape(SEQ_LEN, BATCH, EMBED_DIM).transpose(1, 0, 2)
